# baseline (speedup 1.0000x reference)
import jax
import jax.numpy as jnp
from jax import lax
from jax.experimental import pallas as pl
from jax.experimental.pallas import tpu as pltpu

_VMEM_LIMIT = 64 * 1024 * 1024
_TB = 128


def _round_up(x, m):
    return ((x + m - 1) // m) * m


def _mm(a, w):
    return lax.dot_general(a, w, (((1,), (0,)), ((), ())),
                           preferred_element_type=jnp.float32)


def _band(w, in_w, out_w, stride=1, offset=0, swap=False):
    co, ci = w.shape[0], w.shape[1]
    ows = stride * jnp.arange(out_w)[None, None, None, :]
    oneh = (jnp.arange(in_w)[None, None, :, None] - ows + offset
            - jnp.arange(stride)[:, None, None, None]
            == jnp.arange(5)[None, :, None, None]).astype(w.dtype)
    b = jnp.einsum('ochk,pkiw->phciwo' if swap else 'ochk,pkiw->phciow',
                   w, oneh)
    return b.reshape(stride, 5, ci * in_w, co * out_w)


def _lenet_kernel(x_ref, w1_ref, b1_ref, w2_ref, b2_ref, w5_ref, b5_ref,
                  f6_ref, b6_ref, wo_ref, bo_ref, o_ref, xs_ref, a1_ref,
                  a2_ref):
    tb = o_ref.shape[0]

    xs_ref[0:2] = jnp.zeros((2, tb, 96), jnp.float32)
    xs_ref[2:34] = jnp.transpose(x_ref[...], (2, 0, 1, 3)).reshape(32, tb, 96)
    xs_ref[34:36] = jnp.zeros((2, tb, 96), jnp.float32)

    for oc in range(4):
        base = 8 * oc
        acc0 = _mm(xs_ref[base:base + 8].reshape(8 * tb, 96), w1_ref[0, 0])
        acc1 = _mm(xs_ref[base:base + 8].reshape(8 * tb, 96), w1_ref[1, 0])
        for kh in range(1, 5):
            xs = xs_ref[base + kh:base + kh + 8].reshape(8 * tb, 96)
            acc0 = acc0 + _mm(xs, w1_ref[0, kh])
            acc1 = acc1 + _mm(xs, w1_ref[1, kh])
        t = jnp.maximum(acc0, acc1)
        t = t.reshape(4, 2, tb, 96)
        t = jnp.maximum(t[:, 0], t[:, 1])
        a1_ref[4 * oc:4 * oc + 4] = jnp.maximum(t + b1_ref[...], 0.0)

    for oc in range(2):
        base = 6 * oc
        acc0 = _mm(a1_ref[base:base + 6].reshape(6 * tb, 96), w2_ref[0, 0])
        acc1 = _mm(a1_ref[base:base + 6].reshape(6 * tb, 96), w2_ref[1, 0])
        for kh in range(1, 5):
            xs = a1_ref[base + kh:base + kh + 6].reshape(6 * tb, 96)
            acc0 = acc0 + _mm(xs, w2_ref[0, kh])
            acc1 = acc1 + _mm(xs, w2_ref[1, kh])
        t = jnp.maximum(acc0, acc1)
        t = t.reshape(3, 2, tb, 96)
        t = jnp.maximum(t[:, 0], t[:, 1])
        a2_ref[3 * oc:3 * oc + 3] = jnp.maximum(t + b2_ref[...], 0.0)

    acc = _mm(a2_ref[0:2].reshape(2 * tb, 96), w5_ref[0])
    for kh in range(1, 5):
        acc = acc + _mm(a2_ref[kh:kh + 2].reshape(2 * tb, 96), w5_ref[kh])
    a5 = jnp.maximum(acc + b5_ref[...], 0.0).reshape(2, tb, 240)

    h = _mm(a5[0], f6_ref[0]) + _mm(a5[1], f6_ref[1])
    h = jnp.maximum(h + b6_ref[...], 0.0)

    o_ref[...] = _mm(h, wo_ref[...]) + bo_ref[...]


def kernel(c1_w, c1_b, c3_w, c3_b, c5_wt, c5_b, f6_wt, f6_b, out_wt, out_b, x):
    B = x.shape[0]
    f32 = jnp.float32

    w1b = _band(c1_w.reshape(6, 3, 5, 5), 32, 16, stride=2, offset=2)
    w2b = _band(c3_w.reshape(16, 6, 5, 5), 16, 6, stride=2)
    w5b = _band(c5_wt.T.reshape(120, 16, 5, 5), 6, 2, swap=True)[0]
    b1p = jnp.broadcast_to(c1_b.reshape(6, 1), (6, 16)).reshape(1, 96)
    b2p = jnp.broadcast_to(c3_b.reshape(16, 1), (16, 6)).reshape(1, 96)
    b5t = jnp.concatenate([c5_b.reshape(1, 120)] * 2, axis=1)
    f6c = jnp.stack([jnp.concatenate([f6_wt[0], f6_wt[1]], axis=0),
                     jnp.concatenate([f6_wt[2], f6_wt[3]], axis=0)])
    b6r = f6_b.reshape(1, 84)

    b_pad = _round_up(B, _TB)
    xp = jnp.pad(x, ((0, b_pad - B), (0, 0), (0, 0), (0, 0)))

    nb = b_pad // _TB
    out = pl.pallas_call(
        _lenet_kernel,
        out_shape=jax.ShapeDtypeStruct((b_pad, 128), f32),
        grid=(nb,),
        in_specs=[
            pl.BlockSpec((_TB, 3, 32, 32), lambda i: (i, 0, 0, 0)),
            pl.BlockSpec((2, 5, 96, 96), lambda i: (0, 0, 0, 0)),
            pl.BlockSpec((1, 96), lambda i: (0, 0)),
            pl.BlockSpec((2, 5, 96, 96), lambda i: (0, 0, 0, 0)),
            pl.BlockSpec((1, 96), lambda i: (0, 0)),
            pl.BlockSpec((5, 96, 240), lambda i: (0, 0, 0)),
            pl.BlockSpec((1, 240), lambda i: (0, 0)),
            pl.BlockSpec((2, 240, 84), lambda i: (0, 0, 0)),
            pl.BlockSpec((1, 84), lambda i: (0, 0)),
            pl.BlockSpec((84, 128), lambda i: (0, 0)),
            pl.BlockSpec((1, 128), lambda i: (0, 0)),
        ],
        out_specs=pl.BlockSpec((_TB, 128), lambda i: (i, 0)),
        scratch_shapes=[
            pltpu.VMEM((36, _TB, 96), f32),
            pltpu.VMEM((16, _TB, 96), f32),
            pltpu.VMEM((6, _TB, 96), f32),
        ],
        compiler_params=pltpu.CompilerParams(
            dimension_semantics=("parallel",),
            vmem_limit_bytes=_VMEM_LIMIT),
        cost_estimate=pl.CostEstimate(
            flops=2 * b_pad * (32 * 96 * 192 * 5 + 12 * 96 * 192 * 5
                               + 2 * 96 * 240 * 5 + 2 * 240 * 84 + 84 * 128),
            transcendentals=0,
            bytes_accessed=4 * (32 * b_pad * 96 + b_pad * 128)),
    )(xp, w1b, b1p, w2b, b2p, w5b, b5t, f6c, b6r, out_wt, out_b)
    return out[:B, :10]


# speedup vs baseline: 1.5160x; 1.5160x over previous
import jax
import jax.numpy as jnp
from jax import lax
from jax.experimental import pallas as pl
from jax.experimental.pallas import tpu as pltpu

_VMEM_LIMIT = 64 * 1024 * 1024
_TB = 128


def _round_up(x, m):
    return ((x + m - 1) // m) * m


def _mm(a, w):
    return lax.dot_general(a, w, (((1,), (0,)), ((), ())),
                           preferred_element_type=jnp.float32)


def _band(w, in_w, out_w, offset=0, k_pad=0):
    co, ci = w.shape[0], w.shape[1]
    ows = 2 * jnp.arange(out_w)[None, None, None, :]
    oneh = (jnp.arange(in_w)[None, None, :, None] - ows + offset
            - jnp.arange(2)[:, None, None, None]
            == jnp.arange(5)[None, :, None, None]).astype(w.dtype)
    b = jnp.einsum('ochk,pkiw->phciow', w, oneh)
    b = b.reshape(2, 5, ci * in_w, co * out_w)
    b = jnp.pad(b, ((0, 0), (0, 0), (0, k_pad), (0, 128 - co * out_w)))
    return jnp.transpose(b, (1, 2, 0, 3)).reshape(5, ci * in_w + k_pad, 256)


def _pool_h(acc, rows, tb, b_ref):
    t = jnp.maximum(acc[:, 0:128], acc[:, 128:256])
    t = t.reshape(rows, 2, tb, 128)
    t = jnp.maximum(t[:, 0], t[:, 1])
    return jnp.maximum(t + b_ref[...], 0.0)


def _lenet_kernel(x_ref, w1_ref, b1_ref, w2_ref, b2_ref, w5_ref, b5_ref,
                  f6_ref, b6_ref, wo_ref, bo_ref, o_ref, xs_ref, a1_ref,
                  a2_ref):
    tb = o_ref.shape[0]

    xs_ref[0:2] = jnp.zeros((2, tb, 96), jnp.float32)
    xs_ref[2:34] = x_ref[...]
    xs_ref[34:36] = jnp.zeros((2, tb, 96), jnp.float32)

    for oc in range(4):
        base = 8 * oc
        acc = _mm(xs_ref[base:base + 8].reshape(8 * tb, 96), w1_ref[0])
        for kh in range(1, 5):
            acc = acc + _mm(xs_ref[base + kh:base + kh + 8].reshape(8 * tb, 96),
                            w1_ref[kh])
        a1_ref[4 * oc:4 * oc + 4] = _pool_h(acc, 4, tb, b1_ref)

    for oc in range(2):
        base = 6 * oc
        acc = _mm(a1_ref[base:base + 6].reshape(6 * tb, 128), w2_ref[0])
        for kh in range(1, 5):
            acc = acc + _mm(a1_ref[base + kh:base + kh + 6].reshape(6 * tb, 128),
                            w2_ref[kh])
        a2_ref[3 * oc:3 * oc + 3] = _pool_h(acc, 3, tb, b2_ref)

    acc = _mm(a2_ref[0:2].reshape(2 * tb, 128), w5_ref[0])
    for kh in range(1, 5):
        acc = acc + _mm(a2_ref[kh:kh + 2].reshape(2 * tb, 128), w5_ref[kh])
    a5 = jnp.maximum(acc + b5_ref[...], 0.0).reshape(2, tb, 256)

    h = _mm(a5[0], f6_ref[0]) + _mm(a5[1], f6_ref[1])
    h = jnp.maximum(h + b6_ref[...], 0.0)

    o_ref[...] = _mm(h, wo_ref[...]) + bo_ref[...]


def kernel(c1_w, c1_b, c3_w, c3_b, c5_wt, c5_b, f6_wt, f6_b, out_wt, out_b, x):
    B = x.shape[0]
    f32 = jnp.float32

    w1b = _band(c1_w.reshape(6, 3, 5, 5), 32, 16, offset=2)
    w2b = _band(c3_w.reshape(16, 6, 5, 5), 16, 6, k_pad=32)
    w5 = c5_wt.T.reshape(120, 16, 5, 5)
    oneh5 = (jnp.arange(6)[None, :, None] - jnp.arange(2)[None, None, :]
             == jnp.arange(5)[:, None, None]).astype(f32)
    w5b = jnp.einsum('ochk,kiw->hciwo', w5, oneh5).reshape(5, 96, 240)
    w5b = jnp.pad(w5b, ((0, 0), (0, 32), (0, 16)))
    b1p = jnp.pad(jnp.broadcast_to(c1_b.reshape(6, 1), (6, 16)).reshape(1, 96),
                  ((0, 0), (0, 32)))
    b2p = jnp.pad(jnp.broadcast_to(c3_b.reshape(16, 1), (16, 6)).reshape(1, 96),
                  ((0, 0), (0, 32)))
    b5t = jnp.pad(jnp.concatenate([c5_b.reshape(1, 120)] * 2, axis=1),
                  ((0, 0), (0, 16)))
    f6c = jnp.stack([jnp.concatenate([f6_wt[0], f6_wt[1]], axis=0),
                     jnp.concatenate([f6_wt[2], f6_wt[3]], axis=0)])
    f6c = jnp.pad(f6c, ((0, 0), (0, 16), (0, 0)))
    b6r = f6_b.reshape(1, 84)

    b_pad = _round_up(B, _TB)
    xp = jnp.pad(x, ((0, b_pad - B), (0, 0), (0, 0), (0, 0)))
    xp = jnp.transpose(xp, (2, 0, 1, 3)).reshape(32, b_pad, 96)

    nb = b_pad // _TB
    out = pl.pallas_call(
        _lenet_kernel,
        out_shape=jax.ShapeDtypeStruct((b_pad, 128), f32),
        grid=(nb,),
        in_specs=[
            pl.BlockSpec((32, _TB, 96), lambda i: (0, i, 0)),
            pl.BlockSpec((5, 96, 256), lambda i: (0, 0, 0)),
            pl.BlockSpec((1, 128), lambda i: (0, 0)),
            pl.BlockSpec((5, 128, 256), lambda i: (0, 0, 0)),
            pl.BlockSpec((1, 128), lambda i: (0, 0)),
            pl.BlockSpec((5, 128, 256), lambda i: (0, 0, 0)),
            pl.BlockSpec((1, 256), lambda i: (0, 0)),
            pl.BlockSpec((2, 256, 84), lambda i: (0, 0, 0)),
            pl.BlockSpec((1, 84), lambda i: (0, 0)),
            pl.BlockSpec((84, 128), lambda i: (0, 0)),
            pl.BlockSpec((1, 128), lambda i: (0, 0)),
        ],
        out_specs=pl.BlockSpec((_TB, 128), lambda i: (i, 0)),
        scratch_shapes=[
            pltpu.VMEM((36, _TB, 96), f32),
            pltpu.VMEM((16, _TB, 128), f32),
            pltpu.VMEM((6, _TB, 128), f32),
        ],
        compiler_params=pltpu.CompilerParams(
            dimension_semantics=("parallel",),
            vmem_limit_bytes=_VMEM_LIMIT),
        cost_estimate=pl.CostEstimate(
            flops=2 * b_pad * (32 * 96 * 256 * 5 + 12 * 128 * 256 * 5
                               + 2 * 128 * 256 * 5 + 2 * 256 * 84 + 84 * 128),
            transcendentals=0,
            bytes_accessed=4 * (32 * b_pad * 96 + b_pad * 128)),
    )(xp, w1b, b1p, w2b, b2p, w5b, b5t, f6c, b6r, out_wt, out_b)
    return out[:B, :10]


# speedup vs baseline: 1.7365x; 1.1454x over previous
import jax
import jax.numpy as jnp
from jax import lax
from jax.experimental import pallas as pl
from jax.experimental.pallas import tpu as pltpu

_VMEM_LIMIT = 64 * 1024 * 1024
_TB = 128


def _round_up(x, m):
    return ((x + m - 1) // m) * m


def _mm(a, w):
    return lax.dot_general(a, w, (((1,), (0,)), ((), ())),
                           preferred_element_type=jnp.float32)


def _band(w, in_w, out_w, offset=0, k_pad=0):
    co, ci = w.shape[0], w.shape[1]
    ows = 2 * jnp.arange(out_w)[None, None, None, :]
    oneh = (jnp.arange(in_w)[None, None, :, None] - ows + offset
            - jnp.arange(2)[:, None, None, None]
            == jnp.arange(5)[None, :, None, None]).astype(w.dtype)
    b = jnp.einsum('ochk,pkiw->phciow', w, oneh)
    b = b.reshape(2, 5, ci * in_w, co * out_w)
    b = jnp.pad(b, ((0, 0), (0, 0), (0, k_pad), (0, 128 - co * out_w)))
    return jnp.transpose(b, (1, 2, 0, 3)).reshape(5, ci * in_w + k_pad, 256)


def _pool_h(acc, rows, tb, b_ref):
    t = jnp.maximum(acc[:, 0:128], acc[:, 128:256])
    t = t.reshape(rows, 2, tb, 128)
    t = jnp.maximum(t[:, 0], t[:, 1])
    return jnp.maximum(t + b_ref[...], 0.0)


def _lenet_kernel(x_ref, w1_ref, b1_ref, w2_ref, b2_ref, w5_ref, b5_ref,
                  f6_ref, b6_ref, wo_ref, bo_ref, o_ref, xs_ref, a1_ref,
                  a2_ref):
    tb = o_ref.shape[0]

    xs_ref[0:2] = jnp.zeros((2, tb, 96), jnp.bfloat16)
    xs_ref[2:34] = x_ref[...]
    xs_ref[34:36] = jnp.zeros((2, tb, 96), jnp.bfloat16)

    for oc in range(4):
        base = 8 * oc
        acc = _mm(xs_ref[base:base + 8].reshape(8 * tb, 96), w1_ref[0])
        for kh in range(1, 5):
            acc = acc + _mm(xs_ref[base + kh:base + kh + 8].reshape(8 * tb, 96),
                            w1_ref[kh])
        a1_ref[4 * oc:4 * oc + 4] = _pool_h(acc, 4, tb, b1_ref).astype(jnp.bfloat16)

    for oc in range(2):
        base = 6 * oc
        acc = _mm(a1_ref[base:base + 6].reshape(6 * tb, 128), w2_ref[0])
        for kh in range(1, 5):
            acc = acc + _mm(a1_ref[base + kh:base + kh + 6].reshape(6 * tb, 128),
                            w2_ref[kh])
        a2_ref[3 * oc:3 * oc + 3] = _pool_h(acc, 3, tb, b2_ref).astype(jnp.bfloat16)

    acc = _mm(a2_ref[0:2].reshape(2 * tb, 128), w5_ref[0])
    for kh in range(1, 5):
        acc = acc + _mm(a2_ref[kh:kh + 2].reshape(2 * tb, 128), w5_ref[kh])
    a5 = jnp.maximum(acc + b5_ref[...], 0.0).astype(jnp.bfloat16).reshape(2, tb, 256)

    h = _mm(a5[0], f6_ref[0]) + _mm(a5[1], f6_ref[1])
    h = jnp.maximum(h + b6_ref[...], 0.0).astype(jnp.bfloat16)

    o_ref[...] = _mm(h, wo_ref[...]) + bo_ref[...]


def kernel(c1_w, c1_b, c3_w, c3_b, c5_wt, c5_b, f6_wt, f6_b, out_wt, out_b, x):
    B = x.shape[0]
    f32 = jnp.float32
    bf16 = jnp.bfloat16

    w1b = _band(c1_w.reshape(6, 3, 5, 5), 32, 16, offset=2)
    w2b = _band(c3_w.reshape(16, 6, 5, 5), 16, 6, k_pad=32)
    w5 = c5_wt.T.reshape(120, 16, 5, 5)
    oneh5 = (jnp.arange(6)[None, :, None] - jnp.arange(2)[None, None, :]
             == jnp.arange(5)[:, None, None]).astype(f32)
    w5b = jnp.einsum('ochk,kiw->hciwo', w5, oneh5).reshape(5, 96, 240)
    w5b = jnp.pad(w5b, ((0, 0), (0, 32), (0, 16)))
    b1p = jnp.pad(jnp.broadcast_to(c1_b.reshape(6, 1), (6, 16)).reshape(1, 96),
                  ((0, 0), (0, 32)))
    b2p = jnp.pad(jnp.broadcast_to(c3_b.reshape(16, 1), (16, 6)).reshape(1, 96),
                  ((0, 0), (0, 32)))
    b5t = jnp.pad(jnp.concatenate([c5_b.reshape(1, 120)] * 2, axis=1),
                  ((0, 0), (0, 16)))
    f6c = jnp.stack([jnp.concatenate([f6_wt[0], f6_wt[1]], axis=0),
                     jnp.concatenate([f6_wt[2], f6_wt[3]], axis=0)])
    f6c = jnp.pad(f6c, ((0, 0), (0, 16), (0, 0)))
    b6r = f6_b.reshape(1, 84)

    b_pad = _round_up(B, _TB)
    xp = jnp.pad(x, ((0, b_pad - B), (0, 0), (0, 0), (0, 0)))
    xp = jnp.transpose(xp, (2, 0, 1, 3)).reshape(32, b_pad, 96).astype(bf16)

    w1b, w2b, w5b, f6c = (a.astype(bf16) for a in (w1b, w2b, w5b, f6c))
    wo = out_wt.astype(bf16)

    nb = b_pad // _TB
    out = pl.pallas_call(
        _lenet_kernel,
        out_shape=jax.ShapeDtypeStruct((b_pad, 128), f32),
        grid=(nb,),
        in_specs=[
            pl.BlockSpec((32, _TB, 96), lambda i: (0, i, 0)),
            pl.BlockSpec((5, 96, 256), lambda i: (0, 0, 0)),
            pl.BlockSpec((1, 128), lambda i: (0, 0)),
            pl.BlockSpec((5, 128, 256), lambda i: (0, 0, 0)),
            pl.BlockSpec((1, 128), lambda i: (0, 0)),
            pl.BlockSpec((5, 128, 256), lambda i: (0, 0, 0)),
            pl.BlockSpec((1, 256), lambda i: (0, 0)),
            pl.BlockSpec((2, 256, 84), lambda i: (0, 0, 0)),
            pl.BlockSpec((1, 84), lambda i: (0, 0)),
            pl.BlockSpec((84, 128), lambda i: (0, 0)),
            pl.BlockSpec((1, 128), lambda i: (0, 0)),
        ],
        out_specs=pl.BlockSpec((_TB, 128), lambda i: (i, 0)),
        scratch_shapes=[
            pltpu.VMEM((36, _TB, 96), bf16),
            pltpu.VMEM((16, _TB, 128), bf16),
            pltpu.VMEM((6, _TB, 128), bf16),
        ],
        compiler_params=pltpu.CompilerParams(
            dimension_semantics=("parallel",),
            vmem_limit_bytes=_VMEM_LIMIT),
        cost_estimate=pl.CostEstimate(
            flops=2 * b_pad * (32 * 96 * 256 * 5 + 12 * 128 * 256 * 5
                               + 2 * 128 * 256 * 5 + 2 * 256 * 84 + 84 * 128),
            transcendentals=0,
            bytes_accessed=4 * (32 * b_pad * 96 + b_pad * 128)),
    )(xp, w1b, b1p, w2b, b2p, w5b, b5t, f6c, b6r, wo, out_b)
    return out[:B, :10]


# speedup vs baseline: 1.7939x; 1.0330x over previous
import jax
import jax.numpy as jnp
from jax import lax
from jax.experimental import pallas as pl
from jax.experimental.pallas import tpu as pltpu

_VMEM_LIMIT = 64 * 1024 * 1024
_TB = 256


def _round_up(x, m):
    return ((x + m - 1) // m) * m


def _mm(a, w):
    return lax.dot_general(a, w, (((1,), (0,)), ((), ())),
                           preferred_element_type=jnp.float32)


def _band(w, in_w, out_w, offset=0, k_pad=0):
    co, ci = w.shape[0], w.shape[1]
    ows = 2 * jnp.arange(out_w)[None, None, None, :]
    oneh = (jnp.arange(in_w)[None, None, :, None] - ows + offset
            - jnp.arange(2)[:, None, None, None]
            == jnp.arange(5)[None, :, None, None]).astype(w.dtype)
    b = jnp.einsum('ochk,pkiw->phciow', w, oneh)
    b = b.reshape(2, 5, ci * in_w, co * out_w)
    b = jnp.pad(b, ((0, 0), (0, 0), (0, k_pad), (0, 128 - co * out_w)))
    return jnp.transpose(b, (1, 2, 0, 3)).reshape(5, ci * in_w + k_pad, 256)


def _pool_h(acc, rows, tb, b_ref):
    t = jnp.maximum(acc[:, 0:128], acc[:, 128:256])
    t = t.reshape(rows, 2, tb, 128)
    t = jnp.maximum(t[:, 0], t[:, 1])
    return jnp.maximum(t + b_ref[...], 0.0)


def _lenet_kernel(x_ref, w1_ref, b1_ref, w2_ref, b2_ref, w5_ref, b5_ref,
                  f6_ref, b6_ref, wo_ref, bo_ref, o_ref, xs_ref, a1_ref,
                  a2_ref):
    tb = o_ref.shape[0]

    xs_ref[0:2] = jnp.zeros((2, tb, 96), jnp.bfloat16)
    xs_ref[2:34] = x_ref[...]
    xs_ref[34:36] = jnp.zeros((2, tb, 96), jnp.bfloat16)

    for oc in range(8):
        base = 4 * oc
        acc = _mm(xs_ref[base:base + 4].reshape(4 * tb, 96), w1_ref[0])
        for kh in range(1, 5):
            acc = acc + _mm(xs_ref[base + kh:base + kh + 4].reshape(4 * tb, 96),
                            w1_ref[kh])
        a1_ref[2 * oc:2 * oc + 2] = _pool_h(acc, 2, tb, b1_ref).astype(jnp.bfloat16)

    for oc in range(3):
        base = 4 * oc
        acc = _mm(a1_ref[base:base + 4].reshape(4 * tb, 128), w2_ref[0])
        for kh in range(1, 5):
            acc = acc + _mm(a1_ref[base + kh:base + kh + 4].reshape(4 * tb, 128),
                            w2_ref[kh])
        a2_ref[2 * oc:2 * oc + 2] = _pool_h(acc, 2, tb, b2_ref).astype(jnp.bfloat16)

    acc = _mm(a2_ref[0:2].reshape(2 * tb, 128), w5_ref[0])
    for kh in range(1, 5):
        acc = acc + _mm(a2_ref[kh:kh + 2].reshape(2 * tb, 128), w5_ref[kh])
    a5 = jnp.maximum(acc + b5_ref[...], 0.0).astype(jnp.bfloat16).reshape(2, tb, 256)

    h = _mm(a5[0], f6_ref[0]) + _mm(a5[1], f6_ref[1])
    h = jnp.maximum(h + b6_ref[...], 0.0).astype(jnp.bfloat16)

    o_ref[...] = _mm(h, wo_ref[...]) + bo_ref[...]


def kernel(c1_w, c1_b, c3_w, c3_b, c5_wt, c5_b, f6_wt, f6_b, out_wt, out_b, x):
    B = x.shape[0]
    f32 = jnp.float32
    bf16 = jnp.bfloat16

    w1b = _band(c1_w.reshape(6, 3, 5, 5), 32, 16, offset=2)
    w2b = _band(c3_w.reshape(16, 6, 5, 5), 16, 6, k_pad=32)
    w5 = c5_wt.T.reshape(120, 16, 5, 5)
    oneh5 = (jnp.arange(6)[None, :, None] - jnp.arange(2)[None, None, :]
             == jnp.arange(5)[:, None, None]).astype(f32)
    w5b = jnp.einsum('ochk,kiw->hciwo', w5, oneh5).reshape(5, 96, 240)
    w5b = jnp.pad(w5b, ((0, 0), (0, 32), (0, 16)))
    b1p = jnp.pad(jnp.broadcast_to(c1_b.reshape(6, 1), (6, 16)).reshape(1, 96),
                  ((0, 0), (0, 32)))
    b2p = jnp.pad(jnp.broadcast_to(c3_b.reshape(16, 1), (16, 6)).reshape(1, 96),
                  ((0, 0), (0, 32)))
    b5t = jnp.pad(jnp.concatenate([c5_b.reshape(1, 120)] * 2, axis=1),
                  ((0, 0), (0, 16)))
    f6c = jnp.stack([jnp.concatenate([f6_wt[0], f6_wt[1]], axis=0),
                     jnp.concatenate([f6_wt[2], f6_wt[3]], axis=0)])
    f6c = jnp.pad(f6c, ((0, 0), (0, 16), (0, 0)))
    b6r = f6_b.reshape(1, 84)

    b_pad = _round_up(B, _TB)
    xp = jnp.pad(x, ((0, b_pad - B), (0, 0), (0, 0), (0, 0)))
    xp = jnp.transpose(xp, (2, 0, 1, 3)).reshape(32, b_pad, 96).astype(bf16)

    w1b, w2b, w5b, f6c = (a.astype(bf16) for a in (w1b, w2b, w5b, f6c))
    wo = out_wt.astype(bf16)

    nb = b_pad // _TB
    out = pl.pallas_call(
        _lenet_kernel,
        out_shape=jax.ShapeDtypeStruct((b_pad, 128), f32),
        grid=(nb,),
        in_specs=[
            pl.BlockSpec((32, _TB, 96), lambda i: (0, i, 0)),
            pl.BlockSpec((5, 96, 256), lambda i: (0, 0, 0)),
            pl.BlockSpec((1, 128), lambda i: (0, 0)),
            pl.BlockSpec((5, 128, 256), lambda i: (0, 0, 0)),
            pl.BlockSpec((1, 128), lambda i: (0, 0)),
            pl.BlockSpec((5, 128, 256), lambda i: (0, 0, 0)),
            pl.BlockSpec((1, 256), lambda i: (0, 0)),
            pl.BlockSpec((2, 256, 84), lambda i: (0, 0, 0)),
            pl.BlockSpec((1, 84), lambda i: (0, 0)),
            pl.BlockSpec((84, 128), lambda i: (0, 0)),
            pl.BlockSpec((1, 128), lambda i: (0, 0)),
        ],
        out_specs=pl.BlockSpec((_TB, 128), lambda i: (i, 0)),
        scratch_shapes=[
            pltpu.VMEM((36, _TB, 96), bf16),
            pltpu.VMEM((16, _TB, 128), bf16),
            pltpu.VMEM((6, _TB, 128), bf16),
        ],
        compiler_params=pltpu.CompilerParams(
            dimension_semantics=("parallel",),
            vmem_limit_bytes=_VMEM_LIMIT),
        cost_estimate=pl.CostEstimate(
            flops=2 * b_pad * (32 * 96 * 256 * 5 + 12 * 128 * 256 * 5
                               + 2 * 128 * 256 * 5 + 2 * 256 * 84 + 84 * 128),
            transcendentals=0,
            bytes_accessed=4 * (32 * b_pad * 96 + b_pad * 128)),
    )(xp, w1b, b1p, w2b, b2p, w5b, b5t, f6c, b6r, wo, out_b)
    return out[:B, :10]
